# Initial kernel scaffold; baseline (speedup 1.0000x reference)
import jax
import jax.numpy as jnp
from jax.experimental import pallas as pl


def kernel(node_feats, edge_index, Wp, bp, Wm1, bm1, Wm2, bm2, Wu1, bu1, Wu2, bu2, Wr1, br1, Wr2, br2):
    raise NotImplementedError("write your pallas kernel here")


# speedup vs baseline: 1.5538x; 1.5538x over previous
import functools

import jax
import jax.numpy as jnp
from jax import lax
from jax.experimental import pallas as pl
from jax.experimental.pallas import tpu as pltpu
from jax.experimental.pallas import tpu_sc as plsc

NODES = 50000
EDGES = 800000
HDIM = 64
NLAYER = 3

NCORES = 2
NSUB = 16
NW = NCORES * NSUB
HALF = NODES // NCORES
ROWS_PER_TEC = 1664
ACC_ROWS = NSUB * ROWS_PER_TEC
DUMMY_ROW = ACC_ROWS - 8
CHUNK = 80
EDGES_PER_TEC = EDGES // NSUB
NCHUNKS = EDGES_PER_TEC // CHUNK
WB_CHUNK = 104
GCH = 40
EPW = EDGES // NW
GNCH = EPW // GCH

_MESH = plsc.VectorSubcoreMesh(core_axis_name="c", subcore_axis_name="s")
_SC_PARAMS = pltpu.CompilerParams(use_tc_tiling_on_sc=False,
                                  needs_layout_passes=False)


def _gather_body(a_hbm, b_hbm, src_hbm, dst_hbm, out_hbm,
                 idx_s, idx_d, buf_a, buf_b, sem_a, sem_b):
    c = lax.axis_index("c")
    t = lax.axis_index("s")
    ebase = (t * NCORES + c) * EPW

    @pl.loop(0, GNCH)
    def _chunk(k):
        off = ebase + k * GCH
        pltpu.sync_copy(src_hbm.at[pl.ds(off, GCH)], idx_s)
        pltpu.sync_copy(dst_hbm.at[pl.ds(off, GCH)], idx_d)
        cp_a = pltpu.async_copy(a_hbm.at[idx_s], buf_a, sem_a)
        cp_b = pltpu.async_copy(b_hbm.at[idx_d], buf_b, sem_b)
        cp_a.wait()
        cp_b.wait()

        @pl.loop(0, GCH)
        def _row(r):
            for g in range(HDIM // 16):
                sl = pl.ds(g * 16, 16)
                buf_a[r, sl] = buf_a[r, sl] + buf_b[r, sl]

        pltpu.sync_copy(buf_a, out_hbm.at[pl.ds(off, GCH), :])


_gather_call = functools.partial(
    pl.kernel,
    out_type=jax.ShapeDtypeStruct((EDGES, HDIM), jnp.float32),
    mesh=_MESH,
    compiler_params=_SC_PARAMS,
    scratch_types=[
        pltpu.VMEM((GCH,), jnp.int32),
        pltpu.VMEM((GCH,), jnp.int32),
        pltpu.VMEM((GCH, HDIM), jnp.float32),
        pltpu.VMEM((GCH, HDIM), jnp.float32),
        pltpu.SemaphoreType.DMA,
        pltpu.SemaphoreType.DMA,
    ],
)(_gather_body)


def _scatter_body(m_hbm, dst_hbm, out_hbm,
                  acc, idx_d, idx_c, buf_m, stg, sem_m):
    c = lax.axis_index("c")
    t = lax.axis_index("s")
    base_node = c * HALF
    zero16 = jnp.zeros((16,), jnp.float32)

    @pl.loop(0, WB_CHUNK)
    def _zrow(r):
        for g in range(HDIM // 16):
            stg[r, pl.ds(g * 16, 16)] = zero16

    @pl.loop(0, ROWS_PER_TEC // WB_CHUNK)
    def _zacc(i):
        pltpu.sync_copy(
            stg, acc.at[pl.ds(t * ROWS_PER_TEC + i * WB_CHUNK, WB_CHUNK), :])

    plsc.subcore_barrier()

    ebase = t * EDGES_PER_TEC

    @pl.loop(0, NCHUNKS)
    def _chunk(k):
        off = ebase + k * CHUNK
        pltpu.sync_copy(dst_hbm.at[pl.ds(off, CHUNK)], idx_d)
        cp_m = pltpu.async_copy(m_hbm.at[pl.ds(off, CHUNK), :], buf_m, sem_m)
        for i in range(CHUNK // 16):
            d = idx_d[pl.ds(i * 16, 16)]
            loc = d - base_node
            ok = (loc >= 0) & (loc < HALF)
            idx_c[pl.ds(i * 16, 16)] = jnp.where(ok, loc, DUMMY_ROW)
        cp_m.wait()
        pltpu.sync_copy(buf_m, acc.at[idx_c], add=True)

    plsc.subcore_barrier()

    woff = jnp.minimum(t * ROWS_PER_TEC, HALF - ROWS_PER_TEC)

    @pl.loop(0, ROWS_PER_TEC // WB_CHUNK)
    def _wb(i):
        ro = woff + i * WB_CHUNK
        pltpu.sync_copy(acc.at[pl.ds(ro, WB_CHUNK), :], stg)
        pltpu.sync_copy(stg, out_hbm.at[pl.ds(base_node + ro, WB_CHUNK), :])


_scatter_call = functools.partial(
    pl.kernel,
    out_type=jax.ShapeDtypeStruct((NODES, HDIM), jnp.float32),
    mesh=_MESH,
    compiler_params=_SC_PARAMS,
    scratch_types=[
        pltpu.VMEM_SHARED((ACC_ROWS, HDIM), jnp.float32),
        pltpu.VMEM((CHUNK,), jnp.int32),
        pltpu.VMEM((CHUNK,), jnp.int32),
        pltpu.VMEM((CHUNK, HDIM), jnp.float32),
        pltpu.VMEM((WB_CHUNK, HDIM), jnp.float32),
        pltpu.SemaphoreType.DMA,
    ],
)(_scatter_body)


RB = 400
NBLK = NODES // RB


def _rows(width):
    return pl.BlockSpec((RB, width), lambda i: (i, 0))


def _whole(shape):
    return pl.BlockSpec(shape, lambda i: tuple(0 for _ in shape))


def _bf(x):
    return x.astype(jnp.bfloat16)


def _mm_1x(x, w):
    return jnp.dot(_bf(x), _bf(w), preferred_element_type=jnp.float32)


def _mm_exact(x, w):
    return jnp.dot(x, w, preferred_element_type=jnp.float32,
                   precision=lax.Precision.HIGHEST)


def _silu(x):
    return x * (1.0 / (1.0 + jnp.exp(-x)))


def _pro_body(nf, wp, bp, m1s, m1d, bm1, h_o, a_o, b_o):
    h = _mm_exact(nf[...], wp[...]) + bp[...]
    h_o[...] = h
    a_o[...] = _mm_1x(h, m1s[...])
    b_o[...] = _mm_1x(h, m1d[...]) + bm1[...]


_pro_call = pl.pallas_call(
    _pro_body,
    grid=(NBLK,),
    in_specs=[_rows(8), _whole((8, HDIM)), _whole((1, HDIM)),
              _whole((HDIM, HDIM)), _whole((HDIM, HDIM)), _whole((1, HDIM))],
    out_specs=[_rows(HDIM)] * 3,
    out_shape=[jax.ShapeDtypeStruct((NODES, HDIM), jnp.float32)] * 3,
)


def _node_update(h, agg, u1h, u1a, bu1, wu2, bu2):
    hv = h[...]
    t = _silu(_mm_exact(hv, u1h[...]) + _mm_exact(agg[...], u1a[...])
              + bu1[...])
    return hv + _mm_exact(t, wu2[...]) + bu2[...]


def _mid_body(h, agg, u1h, u1a, bu1, wu2, bu2, m1s, m1d, bm1,
              h_o, a_o, b_o):
    h2 = _node_update(h, agg, u1h, u1a, bu1, wu2, bu2)
    h_o[...] = h2
    a_o[...] = _mm_1x(h2, m1s[...])
    b_o[...] = _mm_1x(h2, m1d[...]) + bm1[...]


_mid_call = pl.pallas_call(
    _mid_body,
    grid=(NBLK,),
    in_specs=[_rows(HDIM), _rows(HDIM),
              _whole((HDIM, HDIM)), _whole((HDIM, HDIM)), _whole((1, HDIM)),
              _whole((HDIM, HDIM)), _whole((1, HDIM)),
              _whole((HDIM, HDIM)), _whole((HDIM, HDIM)), _whole((1, HDIM))],
    out_specs=[_rows(HDIM)] * 3,
    out_shape=[jax.ShapeDtypeStruct((NODES, HDIM), jnp.float32)] * 3,
)


def _fin_body(h, agg, u1h, u1a, bu1, wu2, bu2, wr1, br1, wr2, br2, o):
    h2 = _node_update(h, agg, u1h, u1a, bu1, wu2, bu2)
    r = _silu(_mm_exact(h2, wr1[...]) + br1[...])
    o[...] = _mm_exact(r, wr2[...]) + br2[...]


_fin_call = pl.pallas_call(
    _fin_body,
    grid=(NBLK,),
    in_specs=[_rows(HDIM), _rows(HDIM),
              _whole((HDIM, HDIM)), _whole((HDIM, HDIM)), _whole((1, HDIM)),
              _whole((HDIM, HDIM)), _whole((1, HDIM)),
              _whole((HDIM, HDIM)), _whole((1, HDIM)),
              _whole((HDIM, 1)), _whole((1, 1))],
    out_specs=[_rows(1)],
    out_shape=[jax.ShapeDtypeStruct((NODES, 1), jnp.float32)],
)


def _soft_body(x, o):
    xv = x[...]
    m = jnp.max(xv)
    e = jnp.exp(xv - m)
    o[...] = e / jnp.sum(e)


_soft_call = pl.pallas_call(
    _soft_body,
    out_shape=jax.ShapeDtypeStruct((400, 125), jnp.float32),
)


def kernel(node_feats, edge_index, Wp, bp, Wm1, bm1, Wm2, bm2,
           Wu1, bu1, Wu2, bu2, Wr1, br1, Wr2, br2):
    src = edge_index[0]
    dst = edge_index[1]
    nf8 = jnp.pad(node_feats, ((0, 0), (0, 5)))
    wp8 = jnp.pad(Wp, ((0, 5), (0, 0)))

    h, a, b = _pro_call(nf8, wp8, bp.reshape(1, -1), Wm1[0][:HDIM],
                        Wm1[0][HDIM:], bm1[0].reshape(1, -1))
    logits = None
    for l in range(NLAYER):
        pre = _gather_call(a, b, src, dst)
        m = jax.nn.silu(pre) @ Wm2[l] + bm2[l]
        agg = _scatter_call(m, dst)
        if l + 1 < NLAYER:
            h, a, b = _mid_call(h, agg, Wu1[l][:HDIM], Wu1[l][HDIM:],
                                bu1[l].reshape(1, -1), Wu2[l],
                                bu2[l].reshape(1, -1), Wm1[l + 1][:HDIM],
                                Wm1[l + 1][HDIM:], bm1[l + 1].reshape(1, -1))
        else:
            (logits,) = _fin_call(h, agg, Wu1[l][:HDIM], Wu1[l][HDIM:],
                                  bu1[l].reshape(1, -1), Wu2[l],
                                  bu2[l].reshape(1, -1), Wr1,
                                  br1.reshape(1, -1), Wr2, br2.reshape(1, -1))
    p = _soft_call(logits.reshape(400, 125))
    return p.reshape(NODES)
